# Initial kernel scaffold; baseline (speedup 1.0000x reference)
import jax
import jax.numpy as jnp
from jax.experimental import pallas as pl


def kernel(z_e, codebook):
    raise NotImplementedError("write your pallas kernel here")


# speedup vs baseline: 1.0084x; 1.0084x over previous
import functools

import jax
import jax.numpy as jnp
from jax.experimental import pallas as pl
from jax.experimental.pallas import tpu as pltpu

_D = 64
_K = 1024
_BLK = 512


def _vq_body(z_ref, cb_ref, x2_ref, w2_ref, quant_ref, idx_ref, loss_ref,
             perp_ref, counts_ref, sse_ref, *, n_rows, blk):
    i = pl.program_id(0)
    g = pl.num_programs(0)
    z = z_ref[...]
    cb = cb_ref[...]

    zc = jax.lax.dot_general(z, cb, (((1,), (1,)), ((), ())),
                             preferred_element_type=jnp.float32)
    x2 = x2_ref[...]
    w2 = w2_ref[...]
    d2 = jnp.maximum(x2 - 2.0 * zc + w2, 0.0)
    dist = jnp.sqrt(d2)

    minval = jnp.min(dist, axis=1, keepdims=True)
    lane = jax.lax.broadcasted_iota(jnp.int32, (blk, _K), 1)
    idx = jnp.min(jnp.where(dist == minval, lane, _K), axis=1,
                  keepdims=True)

    onehot = (lane == idx).astype(jnp.float32)
    quant = jax.lax.dot_general(onehot, cb, (((1,), (0,)), ((), ())),
                                preferred_element_type=jnp.float32)
    quant_ref[...] = z + (quant - z)
    idx_ref[...] = idx

    @pl.when(i == 0)
    def _():
        counts_ref[...] = jnp.zeros_like(counts_ref)
        sse_ref[0] = 0.0

    counts_ref[...] += jnp.sum(onehot, axis=0, keepdims=True)
    sse_ref[0] += jnp.sum((z - quant) ** 2)

    @pl.when(i == g - 1)
    def _():
        avg = counts_ref[...] * (1.0 / n_rows)
        ent = jnp.sum(avg * jnp.log(avg + 1e-10))
        perp_ref[0, 0] = jnp.exp(-ent)
        loss_ref[0, 0] = sse_ref[0] / (n_rows * _D) * 0.1


def kernel(z_e, codebook):
    b, e = z_e.shape
    z = z_e.reshape(-1, _D)
    n_rows = z.shape[0]
    blk = _BLK
    grid = n_rows // blk

    x2 = jnp.sum(z * z, axis=1, keepdims=True)
    w2 = jnp.sum(codebook * codebook, axis=1)[None, :]

    quant, idx, loss, perp = pl.pallas_call(
        functools.partial(_vq_body, n_rows=n_rows, blk=blk),
        grid=(grid,),
        in_specs=[
            pl.BlockSpec((blk, _D), lambda i: (i, 0)),
            pl.BlockSpec((_K, _D), lambda i: (0, 0)),
            pl.BlockSpec((blk, 1), lambda i: (i, 0)),
            pl.BlockSpec((1, _K), lambda i: (0, 0)),
        ],
        out_specs=[
            pl.BlockSpec((blk, _D), lambda i: (i, 0)),
            pl.BlockSpec((blk, 1), lambda i: (i, 0)),
            pl.BlockSpec((1, 1), lambda i: (0, 0), memory_space=pltpu.SMEM),
            pl.BlockSpec((1, 1), lambda i: (0, 0), memory_space=pltpu.SMEM),
        ],
        out_shape=[
            jax.ShapeDtypeStruct((n_rows, _D), jnp.float32),
            jax.ShapeDtypeStruct((n_rows, 1), jnp.int32),
            jax.ShapeDtypeStruct((1, 1), jnp.float32),
            jax.ShapeDtypeStruct((1, 1), jnp.float32),
        ],
        scratch_shapes=[
            pltpu.VMEM((1, _K), jnp.float32),
            pltpu.SMEM((1,), jnp.float32),
        ],
    )(z, codebook, x2, w2)

    return (quant.reshape(b, e), loss[0, 0], idx.reshape(b, e // _D),
            perp[0, 0])
